# baseline (speedup 1.0000x reference)
import jax
import jax.numpy as jnp
from jax import lax
from jax.experimental import pallas as pl
from jax.experimental.pallas import tpu as pltpu
from jax.experimental.pallas import tpu_sc as plsc

VOCAB = 1000000
BINS = 100
D = 64
B = 4096
H = 200
NC, NS = 2, 16
NW = NC * NS
RPW = B // NW
G0, G1 = 104, 96
GS = (G0, G1)
OFFS = (0, G0)


def _body(wids, sids, wtab, stab, out, widx_v, sidx_v, rows_v, srows_v,
          obuf_v, gsemw, gsems, ssem):
    wid = lax.axis_index("s") * NC + lax.axis_index("c")
    row0 = wid * RPW
    pltpu.sync_copy(wids.at[pl.ds(row0, RPW)], widx_v)
    pltpu.sync_copy(sids.at[pl.ds(row0, RPW)], sidx_v)

    def start_gathers(i, j):
        g, off = GS[j], OFFS[j]
        pltpu.make_async_copy(
            wtab.at[widx_v.at[i, pl.ds(off, g)]],
            rows_v.at[j, pl.ds(0, g)], gsemw.at[j]).start()
        pltpu.make_async_copy(
            stab.at[sidx_v.at[i, pl.ds(off, g)]],
            srows_v.at[j, pl.ds(0, g)], gsems.at[j]).start()

    def wait_gathers(i, j):
        g, off = GS[j], OFFS[j]
        pltpu.make_async_copy(
            wtab.at[widx_v.at[i, pl.ds(off, g)]],
            rows_v.at[j, pl.ds(0, g)], gsemw.at[j]).wait()
        pltpu.make_async_copy(
            stab.at[sidx_v.at[i, pl.ds(off, g)]],
            srows_v.at[j, pl.ds(0, g)], gsems.at[j]).wait()

    def scatter_desc(i, j):
        g, off = GS[j], OFFS[j]
        return pltpu.make_async_copy(
            obuf_v.at[j, pl.ds(0, g)],
            out.at[row0 + i, pl.ds(off, g)], ssem.at[j])

    for j in range(2):
        start_gathers(0, j)

    @pl.loop(0, RPW)
    def _row(i):
        for j in range(2):
            wait_gathers(i, j)

            @pl.when(i >= 1)
            def _():
                scatter_desc(i - 1, j).wait()

            @pl.loop(0, GS[j], unroll=8)
            def _lk(r):
                for c in range(D // 16):
                    sl = pl.ds(c * 16, 16)
                    obuf_v[j, r, sl] = rows_v[j, r, sl] + srows_v[j, r, sl]

            scatter_desc(i, j).start()

            @pl.when(i + 1 < RPW)
            def _():
                start_gathers(i + 1, j)

    for j in range(2):
        scatter_desc(RPW - 1, j).wait()


@jax.jit
def _sc_embed(wids, sids, wtab, stab):
    kern = pl.kernel(
        _body,
        out_type=jax.ShapeDtypeStruct((B, H, D), jnp.float32),
        mesh=plsc.VectorSubcoreMesh(core_axis_name="c", subcore_axis_name="s"),
        compiler_params=pltpu.CompilerParams(use_tc_tiling_on_sc=False),
        scratch_types=[
            pltpu.VMEM((RPW, H), jnp.int32),
            pltpu.VMEM((RPW, H), jnp.int32),
            pltpu.VMEM((2, G0, D), jnp.float32),
            pltpu.VMEM((2, G0, D), jnp.float32),
            pltpu.VMEM((2, G0, D), jnp.float32),
            pltpu.SemaphoreType.DMA((2,)),
            pltpu.SemaphoreType.DMA((2,)),
            pltpu.SemaphoreType.DMA((2,)),
        ],
    )
    return kern(wids, sids, wtab, stab)


def kernel(input_ids, scores_ids, words_emb, scores_emb):
    return _sc_embed(input_ids.astype(jnp.int32), scores_ids.astype(jnp.int32),
                     words_emb, scores_emb)
